# baseline (speedup 1.0000x reference)
import functools

import jax
import jax.numpy as jnp
from jax import lax
from jax.experimental import pallas as pl
from jax.experimental.pallas import tpu as pltpu
from jax.experimental.pallas import tpu_sc as plsc

B, C, S, F, G = 8, 64, 512, 16, 11
FG = F * G
NV = FG // 16
CH = 32
NW = 32
UPW = (B * C) // NW


def _sc_body(seqs_hbm, seqL_hbm, out_hbm, lbuf, chunk, obuf):
    wid = lax.axis_index("s") * 2 + lax.axis_index("c")
    pltpu.sync_copy(seqL_hbm, lbuf.at[pl.ds(0, B)])

    def unit_body(k, carry):
        u = wid + NW * k
        b = u // C
        L = jnp.minimum(jnp.maximum(lbuf[pl.ds(b, 16)][0], 0), S)
        base = u * (S * FG)
        nchunks = (L + CH - 1) // CH

        init = tuple(jnp.full((16,), -jnp.inf, jnp.float32) for _ in range(NV))

        def chunk_body(i, accs):
            pltpu.sync_copy(
                seqs_hbm.at[pl.ds(base + i * (CH * FG), CH * FG)], chunk)
            valid = L - i * CH
            accs = list(accs)
            for r in range(CH):
                keep = r < valid
                for j in range(NV):
                    row = chunk[pl.ds(r * FG + j * 16, 16)]
                    m = jnp.maximum(accs[j], row)
                    accs[j] = jnp.where(keep, m, accs[j])
            return tuple(accs)

        accs = lax.fori_loop(0, nchunks, chunk_body, init)

        gate = L > 0
        for j in range(NV):
            obuf[pl.ds(j * 16, 16)] = jnp.where(gate, accs[j],
                                                jnp.zeros((16,), jnp.float32))
        pltpu.sync_copy(obuf, out_hbm.at[pl.ds(u * FG, FG)])
        return carry

    lax.fori_loop(0, UPW, unit_body, 0)


@jax.jit
def _pooled(seqs_flat, seqL_flat):
    mesh = plsc.VectorSubcoreMesh(core_axis_name="c", subcore_axis_name="s")
    run = functools.partial(
        pl.kernel,
        mesh=mesh,
        out_type=jax.ShapeDtypeStruct((B * C * FG,), jnp.float32),
        scratch_types=[
            pltpu.VMEM((24,), jnp.int32),
            pltpu.VMEM((CH * FG,), jnp.float32),
            pltpu.VMEM((FG,), jnp.float32),
        ],
    )(_sc_body)
    return run(seqs_flat, seqL_flat)


def kernel(seqs, seqL):
    seqs_flat = seqs.reshape(-1)
    seqL_flat = seqL.reshape(-1).astype(jnp.int32)
    out = _pooled(seqs_flat, seqL_flat)
    return out.reshape(B, C, F, G)


# speedup vs baseline: 1.0136x; 1.0136x over previous
import functools

import jax
import jax.numpy as jnp
from jax import lax
from jax.experimental import pallas as pl
from jax.experimental.pallas import tpu as pltpu
from jax.experimental.pallas import tpu_sc as plsc

B, C, S, F, G = 8, 64, 512, 16, 11
FG = F * G
NV = FG // 16
CH = 32
NW = 32
UPW = (B * C) // NW


def _sc_body(seqs_hbm, seqL_hbm, out_hbm, lbuf, chunk, obuf, sem):
    wid = lax.axis_index("s") * 2 + lax.axis_index("c")
    pltpu.sync_copy(seqL_hbm, lbuf.at[pl.ds(0, B)])

    def unit_body(k, carry):
        u = wid + NW * k
        b = u // C
        L = jnp.minimum(jnp.maximum(lbuf[pl.ds(b, 16)][0], 0), S)
        base = u * (S * FG)
        nchunks = (L + CH - 1) // CH
        nfull = L // CH
        rem = L - nfull * CH

        def fire(i, carry):
            pltpu.make_async_copy(
                seqs_hbm.at[pl.ds(base + i * (CH * FG), CH * FG)],
                chunk.at[pl.ds(i * (CH * FG), CH * FG)],
                sem).start()
            return carry

        lax.fori_loop(0, nchunks, fire, 0)

        def drain(i, carry):
            pltpu.make_async_copy(
                seqs_hbm.at[pl.ds(base, CH * FG)],
                chunk.at[pl.ds(0, CH * FG)],
                sem).wait()
            return carry

        lax.fori_loop(0, nchunks, drain, 0)

        init = tuple(jnp.full((16,), -jnp.inf, jnp.float32) for _ in range(NV))

        def chunk_body(i, accs):
            cbase = i * (CH * FG)
            accs = list(accs)
            for r in range(CH):
                for j in range(NV):
                    row = chunk[pl.ds(cbase + r * FG + j * 16, 16)]
                    accs[j] = jnp.maximum(accs[j], row)
            return tuple(accs)

        accs = lax.fori_loop(0, nfull, chunk_body, init)

        def tail_body(i, accs):
            cbase = nfull * (CH * FG)
            accs = list(accs)
            for r in range(CH):
                keep = r < rem
                for j in range(NV):
                    row = chunk[pl.ds(cbase + r * FG + j * 16, 16)]
                    m = jnp.maximum(accs[j], row)
                    accs[j] = jnp.where(keep, m, accs[j])
            return tuple(accs)

        accs = lax.fori_loop(0, (rem > 0).astype(jnp.int32), tail_body, accs)

        gate = L > 0
        for j in range(NV):
            obuf[pl.ds(j * 16, 16)] = jnp.where(gate, accs[j],
                                                jnp.zeros((16,), jnp.float32))
        pltpu.sync_copy(obuf, out_hbm.at[pl.ds(u * FG, FG)])
        return carry

    lax.fori_loop(0, UPW, unit_body, 0)


@jax.jit
def _pooled(seqs_flat, seqL_flat):
    mesh = plsc.VectorSubcoreMesh(core_axis_name="c", subcore_axis_name="s")
    run = functools.partial(
        pl.kernel,
        mesh=mesh,
        out_type=jax.ShapeDtypeStruct((B * C * FG,), jnp.float32),
        scratch_types=[
            pltpu.VMEM((24,), jnp.int32),
            pltpu.VMEM((S * FG,), jnp.float32),
            pltpu.VMEM((FG,), jnp.float32),
            pltpu.SemaphoreType.DMA,
        ],
    )(_sc_body)
    return run(seqs_flat, seqL_flat)


def kernel(seqs, seqL):
    seqs_flat = seqs.reshape(-1)
    seqL_flat = seqL.reshape(-1).astype(jnp.int32)
    out = _pooled(seqs_flat, seqL_flat)
    return out.reshape(B, C, F, G)


# speedup vs baseline: 15.5089x; 15.3007x over previous
import functools

import jax
import jax.numpy as jnp
from jax import lax
from jax.experimental import pallas as pl
from jax.experimental.pallas import tpu as pltpu
from jax.experimental.pallas import tpu_sc as plsc

B, C, S, F, G = 8, 64, 512, 16, 11
NW = 32
UPW = (B * C) // NW
NT = UPW * G
NEG = float("-inf")


def _sc_body(seqs_hbm, seqL_hbm, out_hbm,
             lbuf, bufA, bufB, pacc, obuf, semA, semB):
    wid = lax.axis_index("s") * 2 + lax.axis_index("c")
    pltpu.sync_copy(seqL_hbm, lbuf.at[pl.ds(0, B)])
    lanes = jnp.arange(16, dtype=jnp.int32)

    def unit_of(t):
        k = t // G
        g = t - k * G
        u = wid + NW * k
        b = u // C
        c = u - b * C
        return b, c, g

    def length_of(b):
        return jnp.minimum(jnp.maximum(lbuf[pl.ds(b, 16)][0], 0), S)

    def fire_panel(t, buf, sem):
        b, c, g = unit_of(t)
        nst = (length_of(b) + 127) // 128

        def fire_st(st, carry):
            for ft in range(2):
                pltpu.make_async_copy(
                    seqs_hbm.at[b, c, g, pl.ds(ft * 8, 8),
                                pl.ds(st * 128, 128)],
                    buf.at[pl.ds(ft * 8, 8), pl.ds(st * 128, 128)],
                    sem).start()
            return carry

        lax.fori_loop(0, nst, fire_st, 0)

    def body(t, buf, sem, nxtbuf, nxtsem):
        b, c, g = unit_of(t)
        L = length_of(b)
        nfull = L // 128
        rem = L - nfull * 128
        nst = nfull + (rem > 0).astype(jnp.int32)

        def fire_next(i, carry):
            fire_panel(t + 1, nxtbuf, nxtsem)
            return carry

        lax.fori_loop(0, (t + 1 < NT).astype(jnp.int32), fire_next, 0)

        def drain(i, carry):
            pltpu.make_async_copy(
                seqs_hbm.at[b, c, g, pl.ds(0, 8), pl.ds(0, 128)],
                buf.at[pl.ds(0, 8), pl.ds(0, 128)],
                sem).wait()
            return carry

        lax.fori_loop(0, 2 * nst, drain, 0)

        init = tuple(jnp.full((16,), NEG, jnp.float32) for _ in range(F))

        def full_st(st, accs):
            s0 = st * 128
            accs = list(accs)
            for f in range(F):
                for sv in range(8):
                    row = buf[f, pl.ds(s0 + sv * 16, 16)]
                    accs[f] = jnp.maximum(accs[f], row)
            return tuple(accs)

        accs = lax.fori_loop(0, nfull, full_st, init)

        def tail_st(i, accs):
            s0 = nfull * 128
            keeps = [(sv * 16 + lanes) < rem for sv in range(8)]
            neg = jnp.full((16,), NEG, jnp.float32)
            accs = list(accs)
            for f in range(F):
                for sv in range(8):
                    row = buf[f, pl.ds(s0 + sv * 16, 16)]
                    accs[f] = jnp.maximum(accs[f],
                                          jnp.where(keeps[sv], row, neg))
            return tuple(accs)

        accs = lax.fori_loop(0, (rem > 0).astype(jnp.int32), tail_st, accs)

        for f in range(F):
            m = accs[f]
            for step in (1, 2, 4, 8):
                perm = jnp.bitwise_xor(lanes, step)
                shuf = lax.gather(
                    m, perm[:, None],
                    dimension_numbers=lax.GatherDimensionNumbers(
                        offset_dims=(), collapsed_slice_dims=(0,),
                        start_index_map=(0,)),
                    slice_sizes=(1,),
                    mode=lax.GatherScatterMode.PROMISE_IN_BOUNDS)
                m = jnp.maximum(m, shuf)
            pacc[pl.ds(g * F + f, 16)] = m

        def assemble(i, carry):
            gate = L > 0
            zero = jnp.zeros((16,), jnp.float32)
            for gg in range(G):
                m = pacc[pl.ds(gg * F, 16)]
                obuf[pl.ds(gg * 16, 16)] = jnp.where(gate, m, zero)
            u = wid + NW * (t // G)
            pltpu.sync_copy(obuf, out_hbm.at[pl.ds(u * (G * F), G * F)])
            return carry

        lax.fori_loop(0, (g == G - 1).astype(jnp.int32), assemble, 0)

    fire_panel(0, bufA, semA)

    def pair(i, carry):
        body(2 * i, bufA, semA, bufB, semB)
        body(2 * i + 1, bufB, semB, bufA, semA)
        return carry

    lax.fori_loop(0, NT // 2, pair, 0)


@jax.jit
def _pooled(seqs_t, seqL_flat):
    mesh = plsc.VectorSubcoreMesh(core_axis_name="c", subcore_axis_name="s")
    run = functools.partial(
        pl.kernel,
        mesh=mesh,
        out_type=jax.ShapeDtypeStruct((B * C * G * F,), jnp.float32),
        scratch_types=[
            pltpu.VMEM((24,), jnp.int32),
            pltpu.VMEM((F, S), jnp.float32),
            pltpu.VMEM((F, S), jnp.float32),
            pltpu.VMEM((G * F + 16,), jnp.float32),
            pltpu.VMEM((G * F,), jnp.float32),
            pltpu.SemaphoreType.DMA,
            pltpu.SemaphoreType.DMA,
        ],
        compiler_params=pltpu.CompilerParams(use_tc_tiling_on_sc=True),
    )(_sc_body)
    return run(seqs_t, seqL_flat)


def kernel(seqs, seqL):
    seqs_t = jnp.transpose(seqs, (0, 1, 4, 3, 2))
    seqL_flat = seqL.reshape(-1).astype(jnp.int32)
    out = _pooled(seqs_t, seqL_flat)
    return out.reshape(B, C, G, F).transpose(0, 1, 3, 2)


# speedup vs baseline: 16.0688x; 1.0361x over previous
import functools

import jax
import jax.numpy as jnp
from jax import lax
from jax.experimental import pallas as pl
from jax.experimental.pallas import tpu as pltpu
from jax.experimental.pallas import tpu_sc as plsc

B, C, S, F, G = 8, 64, 512, 16, 11
NW = 32
UPW = (B * C) // NW
NT = UPW * G
NEG = float("-inf")


def _sc_body(seqs_hbm, seqL_hbm, out_hbm,
             lbuf, bufA, bufB, pacc, obuf, semA, semB):
    wid = lax.axis_index("s") * 2 + lax.axis_index("c")
    pltpu.sync_copy(seqL_hbm, lbuf.at[pl.ds(0, B)])
    lanes = jnp.arange(16, dtype=jnp.int32)

    def unit_of(t):
        k = t // G
        g = t - k * G
        u = wid + NW * k
        b = u // C
        c = u - b * C
        return b, c, g

    def length_of(b):
        return jnp.minimum(jnp.maximum(lbuf[pl.ds(b, 16)][0], 0), S)

    def fire_panel(t, buf, sem):
        b, c, g = unit_of(t)
        nst = (length_of(b) + 127) // 128

        def fire_st(st, carry):
            pltpu.make_async_copy(
                seqs_hbm.at[b, c, g, pl.ds(0, 16), pl.ds(st * 128, 128)],
                buf.at[pl.ds(0, 16), pl.ds(st * 128, 128)],
                sem).start()
            return carry

        lax.fori_loop(0, nst, fire_st, 0)

    def body(t, buf, sem, nxtbuf, nxtsem):
        b, c, g = unit_of(t)
        L = length_of(b)
        nfull = L // 128
        rem = L - nfull * 128
        nst = nfull + (rem > 0).astype(jnp.int32)

        def fire_next(i, carry):
            fire_panel(t + 1, nxtbuf, nxtsem)
            return carry

        lax.fori_loop(0, (t + 1 < NT).astype(jnp.int32), fire_next, 0)

        def drain(i, carry):
            pltpu.make_async_copy(
                seqs_hbm.at[b, c, g, pl.ds(0, 16), pl.ds(0, 128)],
                buf.at[pl.ds(0, 16), pl.ds(0, 128)],
                sem).wait()
            return carry

        lax.fori_loop(0, nst, drain, 0)

        init = tuple(jnp.full((16,), NEG, jnp.float32) for _ in range(F))

        def _tree8(vals):
            while len(vals) > 1:
                vals = [jnp.maximum(vals[i], vals[i + 1])
                        for i in range(0, len(vals) - 1, 2)] + (
                            [vals[-1]] if len(vals) % 2 else [])
            return vals[0]

        def full_st(st, accs):
            s0 = st * 128
            accs = list(accs)
            for f in range(F):
                rows = [buf[f, pl.ds(s0 + sv * 16, 16)] for sv in range(8)]
                accs[f] = jnp.maximum(accs[f], _tree8(rows))
            return tuple(accs)

        accs = lax.fori_loop(0, nfull, full_st, init)

        def tail_st(i, accs):
            s0 = nfull * 128
            keeps = [(sv * 16 + lanes) < rem for sv in range(8)]
            neg = jnp.full((16,), NEG, jnp.float32)
            accs = list(accs)
            for f in range(F):
                rows = [jnp.where(keeps[sv],
                                  buf[f, pl.ds(s0 + sv * 16, 16)], neg)
                        for sv in range(8)]
                accs[f] = jnp.maximum(accs[f], _tree8(rows))
            return tuple(accs)

        accs = lax.fori_loop(0, (rem > 0).astype(jnp.int32), tail_st, accs)

        for f in range(F):
            m = accs[f]
            for step in (1, 2, 4, 8):
                perm = jnp.bitwise_xor(lanes, step)
                shuf = lax.gather(
                    m, perm[:, None],
                    dimension_numbers=lax.GatherDimensionNumbers(
                        offset_dims=(), collapsed_slice_dims=(0,),
                        start_index_map=(0,)),
                    slice_sizes=(1,),
                    mode=lax.GatherScatterMode.PROMISE_IN_BOUNDS)
                m = jnp.maximum(m, shuf)
            pacc[pl.ds(g * F + f, 16)] = m

        def assemble(i, carry):
            gate = L > 0
            zero = jnp.zeros((16,), jnp.float32)
            for gg in range(G):
                m = pacc[pl.ds(gg * F, 16)]
                obuf[pl.ds(gg * 16, 16)] = jnp.where(gate, m, zero)
            u = wid + NW * (t // G)
            pltpu.sync_copy(obuf, out_hbm.at[pl.ds(u * (G * F), G * F)])
            return carry

        lax.fori_loop(0, (g == G - 1).astype(jnp.int32), assemble, 0)

    fire_panel(0, bufA, semA)

    def pair(i, carry):
        body(2 * i, bufA, semA, bufB, semB)
        body(2 * i + 1, bufB, semB, bufA, semA)
        return carry

    lax.fori_loop(0, NT // 2, pair, 0)


@jax.jit
def _pooled(seqs_t, seqL_flat):
    mesh = plsc.VectorSubcoreMesh(core_axis_name="c", subcore_axis_name="s")
    run = functools.partial(
        pl.kernel,
        mesh=mesh,
        out_type=jax.ShapeDtypeStruct((B * C * G * F,), jnp.float32),
        scratch_types=[
            pltpu.VMEM((24,), jnp.int32),
            pltpu.VMEM((F, S), jnp.float32),
            pltpu.VMEM((F, S), jnp.float32),
            pltpu.VMEM((G * F + 16,), jnp.float32),
            pltpu.VMEM((G * F,), jnp.float32),
            pltpu.SemaphoreType.DMA,
            pltpu.SemaphoreType.DMA,
        ],
        compiler_params=pltpu.CompilerParams(use_tc_tiling_on_sc=True),
    )(_sc_body)
    return run(seqs_t, seqL_flat)


def kernel(seqs, seqL):
    seqs_t = jnp.transpose(seqs, (0, 1, 4, 3, 2))
    seqL_flat = seqL.reshape(-1).astype(jnp.int32)
    out = _pooled(seqs_t, seqL_flat)
    return out.reshape(B, C, G, F).transpose(0, 1, 3, 2)
